# Initial kernel scaffold; baseline (speedup 1.0000x reference)
import jax
import jax.numpy as jnp
from jax.experimental import pallas as pl


def kernel(x, edge_index, batch, eps0, W1_0, b1_0, W2_0, b2_0, g0, be0, eps1, W1_1, b1_1, W2_1, b2_1, g1, be1, eps2, W1_2, b1_2, W2_2, b2_2, g2, be2, Wm, bm, Wv, bv):
    raise NotImplementedError("write your pallas kernel here")


# speedup vs baseline: 4.6143x; 4.6143x over previous
import functools

import jax
import jax.numpy as jnp
from jax import lax
from jax.experimental import pallas as pl
from jax.experimental.pallas import tpu as pltpu
from jax.experimental.pallas import tpu_sc as plsc

N = 10000
E = 320000
D = 128
HID = 128
L = 64
G = 64

NC = 2
NS = 16
NW = NC * NS
EPW = E // NW
CH = 80
NCHUNK = EPW // CH
RPT = 624
TAIL = N - NS * RPT


def _edge_messages(h, src, dst, zeros):
    mesh = plsc.VectorSubcoreMesh(
        core_axis_name="c", subcore_axis_name="s",
        num_cores=NC, num_subcores=NS)

    @functools.partial(
        pl.kernel,
        out_type=jax.ShapeDtypeStruct((NC * N, D), jnp.float32),
        mesh=mesh,
        scratch_types=[
            pltpu.VMEM((CH,), jnp.int32),
            pltpu.VMEM((CH,), jnp.int32),
            pltpu.VMEM((CH, D), jnp.float32),
            pltpu.VMEM_SHARED((N, D), jnp.float32),
            pltpu.SemaphoreType.DMA,
        ],
    )
    def k(h_hbm, src_hbm, dst_hbm, z_hbm, out_hbm, sidx, didx, rows, acc, sem):
        c = lax.axis_index("c")
        s = lax.axis_index("s")
        wid = c * NS + s
        pltpu.sync_copy(z_hbm.at[pl.ds(s * RPT, RPT)],
                        acc.at[pl.ds(s * RPT, RPT)])

        @pl.when(s == NS - 1)
        def _zero_tail():
            pltpu.sync_copy(z_hbm.at[pl.ds(NS * RPT, TAIL)],
                            acc.at[pl.ds(NS * RPT, TAIL)])

        plsc.subcore_barrier()
        base = wid * EPW

        @pl.loop(0, NCHUNK)
        def _chunk(i):
            off = pl.multiple_of(base + i * CH, 8)
            pltpu.sync_copy(src_hbm.at[pl.ds(off, CH)], sidx)
            pltpu.sync_copy(dst_hbm.at[pl.ds(off, CH)], didx)
            pltpu.async_copy(h_hbm.at[sidx], rows, sem).wait()
            pltpu.sync_copy(rows, acc.at[didx], add=True)

        plsc.subcore_barrier()
        pltpu.sync_copy(acc.at[pl.ds(s * RPT, RPT)],
                        out_hbm.at[pl.ds(c * N + s * RPT, RPT)])

        @pl.when(s == NS - 1)
        def _drain_tail():
            pltpu.sync_copy(acc.at[pl.ds(NS * RPT, TAIL)],
                            out_hbm.at[pl.ds(c * N + NS * RPT, TAIL)])

    return k(h, src, dst, zeros)


def _gin_layer_tc(h, msg2, eps, W1, b1, W2, b2, g, be):
    def body(eps_ref, h_ref, m_ref, w1_ref, b1_ref, w2_ref, b2_ref,
             g_ref, be_ref, o_ref):
        e = eps_ref[0]
        z = (1.0 + e) * h_ref[...] + m_ref[0] + m_ref[1]
        a = jnp.maximum(
            jnp.dot(z, w1_ref[...], preferred_element_type=jnp.float32)
            + b1_ref[...], 0.0)
        u = (jnp.dot(a, w2_ref[...], preferred_element_type=jnp.float32)
             + b2_ref[...])
        mu = jnp.mean(u, axis=0, keepdims=True)
        dvc = u - mu
        var = jnp.mean(dvc * dvc, axis=0, keepdims=True)
        o_ref[...] = jnp.maximum(
            dvc * lax.rsqrt(var + 1e-5) * g_ref[...] + be_ref[...], 0.0)

    return pl.pallas_call(
        body,
        out_shape=jax.ShapeDtypeStruct((N, HID), jnp.float32),
        in_specs=[pl.BlockSpec(memory_space=pltpu.SMEM)]
        + [pl.BlockSpec(memory_space=pltpu.VMEM)] * 8,
    )(eps.reshape(1), h, msg2, W1, b1.reshape(1, HID), W2,
      b2.reshape(1, HID), g.reshape(1, HID), be.reshape(1, HID))


def _pool_heads_tc(h, batchf, Wm, bm, Wv, bv):
    def body(h_ref, b_ref, wm_ref, bm_ref, wv_ref, bv_ref, om_ref, ov_ref):
        iota_g = lax.broadcasted_iota(jnp.int32, (1, G), 1).astype(jnp.float32)
        oh = (b_ref[...] == iota_g).astype(jnp.float32)
        ones_col = jnp.ones((N, 1), dtype=jnp.float32)
        cnt = lax.dot_general(oh, ones_col, (((0,), (0,)), ((), ())),
                              preferred_element_type=jnp.float32)
        hgs = lax.dot_general(oh, h_ref[...], (((0,), (0,)), ((), ())),
                              preferred_element_type=jnp.float32)
        hg = hgs * (1.0 / jnp.clip(cnt, 1.0))
        om_ref[...] = (jnp.dot(hg, wm_ref[...],
                               preferred_element_type=jnp.float32)
                       + bm_ref[...])
        ov_ref[...] = (jnp.dot(hg, wv_ref[...],
                               preferred_element_type=jnp.float32)
                       + bv_ref[...])

    return pl.pallas_call(
        body,
        out_shape=(jax.ShapeDtypeStruct((G, L), jnp.float32),
                   jax.ShapeDtypeStruct((G, L), jnp.float32)),
    )(h, batchf, Wm, bm.reshape(1, L), Wv, bv.reshape(1, L))


def kernel(x, edge_index, batch,
           eps0, W1_0, b1_0, W2_0, b2_0, g0, be0,
           eps1, W1_1, b1_1, W2_1, b2_1, g1, be1,
           eps2, W1_2, b1_2, W2_2, b2_2, g2, be2,
           Wm, bm, Wv, bv):
    src = edge_index[0]
    dst = edge_index[1]
    zeros = jnp.zeros((N, D), dtype=jnp.float32)
    batchf = batch.astype(jnp.float32).reshape(N, 1)

    h = x
    for eps, W1, b1, W2, b2, g, be in (
            (eps0, W1_0, b1_0, W2_0, b2_0, g0, be0),
            (eps1, W1_1, b1_1, W2_1, b2_1, g1, be1),
            (eps2, W1_2, b1_2, W2_2, b2_2, g2, be2)):
        msg2 = _edge_messages(h, src, dst, zeros).reshape(NC, N, D)
        h = _gin_layer_tc(h, msg2, eps, W1, b1, W2, b2, g, be)

    return _pool_heads_tc(h, batchf, Wm, bm, Wv, bv)


# speedup vs baseline: 9.3822x; 2.0333x over previous
import functools

import jax
import jax.numpy as jnp
from jax import lax
from jax.experimental import pallas as pl
from jax.experimental.pallas import tpu as pltpu
from jax.experimental.pallas import tpu_sc as plsc

N = 10000
E = 320000
D = 128
HID = 128
L = 64
G = 64

NC = 2
NS = 16
NW = NC * NS
EPW = E // NW
CH = 40
NCHUNK = EPW // CH
NBUF = 5
GA = 2
IA = 4
RPT = 624
TAIL = N - NS * RPT


def _edge_messages(h, src, dst, zeros):
    mesh = plsc.VectorSubcoreMesh(
        core_axis_name="c", subcore_axis_name="s",
        num_cores=NC, num_subcores=NS)

    @functools.partial(
        pl.kernel,
        out_type=jax.ShapeDtypeStruct((NC * N, D), jnp.float32),
        mesh=mesh,
        scratch_types=[
            pltpu.VMEM((NBUF, CH), jnp.int32),
            pltpu.VMEM((NBUF, CH), jnp.int32),
            pltpu.VMEM((NBUF, CH, D), jnp.float32),
            pltpu.VMEM_SHARED((N, D), jnp.float32),
        ] + [pltpu.SemaphoreType.DMA] * (3 * NBUF),
    )
    def k(h_hbm, src_hbm, dst_hbm, z_hbm, out_hbm, sidx, didx, rows, acc,
          is0, is1, is2, is3, is4,
          gs0, gs1, gs2, gs3, gs4,
          ss0, ss1, ss2, ss3, ss4):
        isems = (is0, is1, is2, is3, is4)
        gsems = (gs0, gs1, gs2, gs3, gs4)
        ssems = (ss0, ss1, ss2, ss3, ss4)
        c = lax.axis_index("c")
        s = lax.axis_index("s")
        wid = c * NS + s
        base = wid * EPW

        def idx_start(i, b):
            off = pl.multiple_of(base + i * CH, 8)
            pltpu.async_copy(src_hbm.at[pl.ds(off, CH)], sidx.at[b], isems[b])
            pltpu.async_copy(dst_hbm.at[pl.ds(off, CH)], didx.at[b], isems[b])

        def idx_wait(b):
            pltpu.make_async_copy(src_hbm.at[pl.ds(0, CH)], sidx.at[b],
                                  isems[b]).wait()
            pltpu.make_async_copy(dst_hbm.at[pl.ds(0, CH)], didx.at[b],
                                  isems[b]).wait()

        pltpu.sync_copy(z_hbm.at[pl.ds(s * RPT, RPT)],
                        acc.at[pl.ds(s * RPT, RPT)])

        @pl.when(s == NS - 1)
        def _zero_tail():
            pltpu.sync_copy(z_hbm.at[pl.ds(NS * RPT, TAIL)],
                            acc.at[pl.ds(NS * RPT, TAIL)])

        plsc.subcore_barrier()

        for b in range(IA):
            idx_start(b, b)
        for b in range(GA):
            idx_wait(b)
            pltpu.async_copy(h_hbm.at[sidx.at[b]], rows.at[b], gsems[b])

        @pl.loop(0, NCHUNK // NBUF)
        def _grp(g):
            i0 = g * NBUF
            for b in range(NBUF):
                i = i0 + b
                pltpu.make_async_copy(h_hbm.at[sidx.at[b]], rows.at[b],
                                      gsems[b]).wait()
                pltpu.async_copy(rows.at[b], acc.at[didx.at[b]], ssems[b],
                                 add=True)
                bg = (b + GA) % NBUF

                @pl.when(i + GA < NCHUNK)
                def _issue_gather():
                    idx_wait(bg)

                    @pl.when(i >= NBUF - GA)
                    def _wait_prev_scatter():
                        pltpu.make_async_copy(rows.at[bg], acc.at[didx.at[bg]],
                                              ssems[bg]).wait()

                    pltpu.async_copy(h_hbm.at[sidx.at[bg]], rows.at[bg],
                                     gsems[bg])

                bi = (b + IA) % NBUF

                @pl.when(i + IA < NCHUNK)
                def _issue_idx():
                    idx_start(i + IA, bi)

        for b in range(NBUF):
            pltpu.make_async_copy(rows.at[b], acc.at[didx.at[b]],
                                  ssems[b]).wait()

        plsc.subcore_barrier()
        pltpu.sync_copy(acc.at[pl.ds(s * RPT, RPT)],
                        out_hbm.at[pl.ds(c * N + s * RPT, RPT)])

        @pl.when(s == NS - 1)
        def _drain_tail():
            pltpu.sync_copy(acc.at[pl.ds(NS * RPT, TAIL)],
                            out_hbm.at[pl.ds(c * N + NS * RPT, TAIL)])

    return k(h, src, dst, zeros)


def _gin_layer_tc(h, msg2, eps, W1, b1, W2, b2, g, be):
    def body(eps_ref, h_ref, m_ref, w1_ref, b1_ref, w2_ref, b2_ref,
             g_ref, be_ref, o_ref):
        e = eps_ref[0]
        z = (1.0 + e) * h_ref[...] + m_ref[0] + m_ref[1]
        a = jnp.maximum(
            jnp.dot(z, w1_ref[...], preferred_element_type=jnp.float32)
            + b1_ref[...], 0.0)
        u = (jnp.dot(a, w2_ref[...], preferred_element_type=jnp.float32)
             + b2_ref[...])
        mu = jnp.mean(u, axis=0, keepdims=True)
        dvc = u - mu
        var = jnp.mean(dvc * dvc, axis=0, keepdims=True)
        o_ref[...] = jnp.maximum(
            dvc * lax.rsqrt(var + 1e-5) * g_ref[...] + be_ref[...], 0.0)

    return pl.pallas_call(
        body,
        out_shape=jax.ShapeDtypeStruct((N, HID), jnp.float32),
        in_specs=[pl.BlockSpec(memory_space=pltpu.SMEM)]
        + [pl.BlockSpec(memory_space=pltpu.VMEM)] * 8,
    )(eps.reshape(1), h, msg2, W1, b1.reshape(1, HID), W2,
      b2.reshape(1, HID), g.reshape(1, HID), be.reshape(1, HID))


def _pool_heads_tc(h, batchf, Wm, bm, Wv, bv):
    def body(h_ref, b_ref, wm_ref, bm_ref, wv_ref, bv_ref, om_ref, ov_ref):
        iota_g = lax.broadcasted_iota(jnp.int32, (1, G), 1).astype(jnp.float32)
        oh = (b_ref[...] == iota_g).astype(jnp.float32)
        ones_col = jnp.ones((N, 1), dtype=jnp.float32)
        cnt = lax.dot_general(oh, ones_col, (((0,), (0,)), ((), ())),
                              preferred_element_type=jnp.float32)
        hgs = lax.dot_general(oh, h_ref[...], (((0,), (0,)), ((), ())),
                              preferred_element_type=jnp.float32)
        hg = hgs * (1.0 / jnp.clip(cnt, 1.0))
        om_ref[...] = (jnp.dot(hg, wm_ref[...],
                               preferred_element_type=jnp.float32)
                       + bm_ref[...])
        ov_ref[...] = (jnp.dot(hg, wv_ref[...],
                               preferred_element_type=jnp.float32)
                       + bv_ref[...])

    return pl.pallas_call(
        body,
        out_shape=(jax.ShapeDtypeStruct((G, L), jnp.float32),
                   jax.ShapeDtypeStruct((G, L), jnp.float32)),
    )(h, batchf, Wm, bm.reshape(1, L), Wv, bv.reshape(1, L))


def kernel(x, edge_index, batch,
           eps0, W1_0, b1_0, W2_0, b2_0, g0, be0,
           eps1, W1_1, b1_1, W2_1, b2_1, g1, be1,
           eps2, W1_2, b1_2, W2_2, b2_2, g2, be2,
           Wm, bm, Wv, bv):
    src = edge_index[0]
    dst = edge_index[1]
    zeros = jnp.zeros((N, D), dtype=jnp.float32)
    batchf = batch.astype(jnp.float32).reshape(N, 1)

    h = x
    for eps, W1, b1, W2, b2, g, be in (
            (eps0, W1_0, b1_0, W2_0, b2_0, g0, be0),
            (eps1, W1_1, b1_1, W2_1, b2_1, g1, be1),
            (eps2, W1_2, b1_2, W2_2, b2_2, g2, be2)):
        msg2 = _edge_messages(h, src, dst, zeros).reshape(NC, N, D)
        h = _gin_layer_tc(h, msg2, eps, W1, b1, W2, b2, g, be)

    return _pool_heads_tc(h, batchf, Wm, bm, Wv, bv)


# speedup vs baseline: 9.6046x; 1.0237x over previous
import functools

import jax
import jax.numpy as jnp
from jax import lax
from jax.experimental import pallas as pl
from jax.experimental.pallas import tpu as pltpu
from jax.experimental.pallas import tpu_sc as plsc

N = 10000
E = 320000
D = 128
HID = 128
L = 64
G = 64

NC = 2
NS = 16
NW = NC * NS
EPW = E // NW
CH = 40
NCHUNK = EPW // CH
NBUF = 5
GA = 3
IA = 4
RPT = 624
TAIL = N - NS * RPT


def _edge_messages(h, src, dst, zeros):
    mesh = plsc.VectorSubcoreMesh(
        core_axis_name="c", subcore_axis_name="s",
        num_cores=NC, num_subcores=NS)

    @functools.partial(
        pl.kernel,
        out_type=jax.ShapeDtypeStruct((NC * N, D), jnp.float32),
        mesh=mesh,
        scratch_types=[
            pltpu.VMEM((NBUF, CH), jnp.int32),
            pltpu.VMEM((NBUF, CH), jnp.int32),
            pltpu.VMEM((NBUF, CH, D), jnp.float32),
            pltpu.VMEM_SHARED((N, D), jnp.float32),
        ] + [pltpu.SemaphoreType.DMA] * (3 * NBUF),
    )
    def k(h_hbm, src_hbm, dst_hbm, z_hbm, out_hbm, sidx, didx, rows, acc,
          is0, is1, is2, is3, is4,
          gs0, gs1, gs2, gs3, gs4,
          ss0, ss1, ss2, ss3, ss4):
        isems = (is0, is1, is2, is3, is4)
        gsems = (gs0, gs1, gs2, gs3, gs4)
        ssems = (ss0, ss1, ss2, ss3, ss4)
        c = lax.axis_index("c")
        s = lax.axis_index("s")
        wid = c * NS + s
        base = wid * EPW

        def idx_start(i, b):
            off = pl.multiple_of(base + i * CH, 8)
            pltpu.async_copy(src_hbm.at[pl.ds(off, CH)], sidx.at[b], isems[b])
            pltpu.async_copy(dst_hbm.at[pl.ds(off, CH)], didx.at[b], isems[b])

        def idx_wait(b):
            pltpu.make_async_copy(src_hbm.at[pl.ds(0, CH)], sidx.at[b],
                                  isems[b]).wait()
            pltpu.make_async_copy(dst_hbm.at[pl.ds(0, CH)], didx.at[b],
                                  isems[b]).wait()

        pltpu.sync_copy(z_hbm.at[pl.ds(s * RPT, RPT)],
                        acc.at[pl.ds(s * RPT, RPT)])

        @pl.when(s == NS - 1)
        def _zero_tail():
            pltpu.sync_copy(z_hbm.at[pl.ds(NS * RPT, TAIL)],
                            acc.at[pl.ds(NS * RPT, TAIL)])

        plsc.subcore_barrier()

        for b in range(IA):
            idx_start(b, b)
        for b in range(GA):
            idx_wait(b)
            pltpu.async_copy(h_hbm.at[sidx.at[b]], rows.at[b], gsems[b])

        @pl.loop(0, NCHUNK // NBUF)
        def _grp(g):
            i0 = g * NBUF
            for b in range(NBUF):
                i = i0 + b
                pltpu.make_async_copy(h_hbm.at[sidx.at[b]], rows.at[b],
                                      gsems[b]).wait()
                pltpu.async_copy(rows.at[b], acc.at[didx.at[b]], ssems[b],
                                 add=True)
                bg = (b + GA) % NBUF

                @pl.when(i + GA < NCHUNK)
                def _issue_gather():
                    idx_wait(bg)

                    @pl.when(i >= NBUF - GA)
                    def _wait_prev_scatter():
                        pltpu.make_async_copy(rows.at[bg], acc.at[didx.at[bg]],
                                              ssems[bg]).wait()

                    pltpu.async_copy(h_hbm.at[sidx.at[bg]], rows.at[bg],
                                     gsems[bg])

                bi = (b + IA) % NBUF

                @pl.when(i + IA < NCHUNK)
                def _issue_idx():
                    idx_start(i + IA, bi)

        for b in range(NBUF):
            pltpu.make_async_copy(rows.at[b], acc.at[didx.at[b]],
                                  ssems[b]).wait()

        plsc.subcore_barrier()
        pltpu.sync_copy(acc.at[pl.ds(s * RPT, RPT)],
                        out_hbm.at[pl.ds(c * N + s * RPT, RPT)])

        @pl.when(s == NS - 1)
        def _drain_tail():
            pltpu.sync_copy(acc.at[pl.ds(NS * RPT, TAIL)],
                            out_hbm.at[pl.ds(c * N + NS * RPT, TAIL)])

    return k(h, src, dst, zeros)


def _gin_layer_tc(h, msg2, eps, W1, b1, W2, b2, g, be):
    def body(eps_ref, h_ref, m_ref, w1_ref, b1_ref, w2_ref, b2_ref,
             g_ref, be_ref, o_ref):
        e = eps_ref[0]
        z = (1.0 + e) * h_ref[...] + m_ref[0] + m_ref[1]
        a = jnp.maximum(
            jnp.dot(z, w1_ref[...], preferred_element_type=jnp.float32)
            + b1_ref[...], 0.0)
        u = (jnp.dot(a, w2_ref[...], preferred_element_type=jnp.float32)
             + b2_ref[...])
        mu = jnp.mean(u, axis=0, keepdims=True)
        dvc = u - mu
        var = jnp.mean(dvc * dvc, axis=0, keepdims=True)
        o_ref[...] = jnp.maximum(
            dvc * lax.rsqrt(var + 1e-5) * g_ref[...] + be_ref[...], 0.0)

    return pl.pallas_call(
        body,
        out_shape=jax.ShapeDtypeStruct((N, HID), jnp.float32),
        in_specs=[pl.BlockSpec(memory_space=pltpu.SMEM)]
        + [pl.BlockSpec(memory_space=pltpu.VMEM)] * 8,
    )(eps.reshape(1), h, msg2, W1, b1.reshape(1, HID), W2,
      b2.reshape(1, HID), g.reshape(1, HID), be.reshape(1, HID))


def _pool_heads_tc(h, batchf, Wm, bm, Wv, bv):
    def body(h_ref, b_ref, wm_ref, bm_ref, wv_ref, bv_ref, om_ref, ov_ref):
        iota_g = lax.broadcasted_iota(jnp.int32, (1, G), 1).astype(jnp.float32)
        oh = (b_ref[...] == iota_g).astype(jnp.float32)
        ones_col = jnp.ones((N, 1), dtype=jnp.float32)
        cnt = lax.dot_general(oh, ones_col, (((0,), (0,)), ((), ())),
                              preferred_element_type=jnp.float32)
        hgs = lax.dot_general(oh, h_ref[...], (((0,), (0,)), ((), ())),
                              preferred_element_type=jnp.float32)
        hg = hgs * (1.0 / jnp.clip(cnt, 1.0))
        om_ref[...] = (jnp.dot(hg, wm_ref[...],
                               preferred_element_type=jnp.float32)
                       + bm_ref[...])
        ov_ref[...] = (jnp.dot(hg, wv_ref[...],
                               preferred_element_type=jnp.float32)
                       + bv_ref[...])

    return pl.pallas_call(
        body,
        out_shape=(jax.ShapeDtypeStruct((G, L), jnp.float32),
                   jax.ShapeDtypeStruct((G, L), jnp.float32)),
    )(h, batchf, Wm, bm.reshape(1, L), Wv, bv.reshape(1, L))


def kernel(x, edge_index, batch,
           eps0, W1_0, b1_0, W2_0, b2_0, g0, be0,
           eps1, W1_1, b1_1, W2_1, b2_1, g1, be1,
           eps2, W1_2, b1_2, W2_2, b2_2, g2, be2,
           Wm, bm, Wv, bv):
    src = edge_index[0]
    dst = edge_index[1]
    zeros = jnp.zeros((N, D), dtype=jnp.float32)
    batchf = batch.astype(jnp.float32).reshape(N, 1)

    h = x
    for eps, W1, b1, W2, b2, g, be in (
            (eps0, W1_0, b1_0, W2_0, b2_0, g0, be0),
            (eps1, W1_1, b1_1, W2_1, b2_1, g1, be1),
            (eps2, W1_2, b1_2, W2_2, b2_2, g2, be2)):
        msg2 = _edge_messages(h, src, dst, zeros).reshape(NC, N, D)
        h = _gin_layer_tc(h, msg2, eps, W1, b1, W2, b2, g, be)

    return _pool_heads_tc(h, batchf, Wm, bm, Wv, bv)


# speedup vs baseline: 12.4807x; 1.2995x over previous
import functools

import jax
import jax.numpy as jnp
from jax import lax
from jax.experimental import pallas as pl
from jax.experimental.pallas import tpu as pltpu
from jax.experimental.pallas import tpu_sc as plsc

N = 10000
E = 320000
D = 128
HID = 128
L = 64
G = 64

NC = 2
NS = 16
NW = NC * NS
EPW = E // NW
CH = 80
NCHUNK = EPW // CH
NBUF = 4
GA = 2
IA = 3
RPT = 624
TAIL = N - NS * RPT


def _edge_messages(h, src, dst, zeros):
    mesh = plsc.VectorSubcoreMesh(
        core_axis_name="c", subcore_axis_name="s",
        num_cores=NC, num_subcores=NS)

    @functools.partial(
        pl.kernel,
        out_type=jax.ShapeDtypeStruct((NC * N, D), jnp.float32),
        mesh=mesh,
        scratch_types=[
            pltpu.VMEM((NBUF, CH), jnp.int32),
            pltpu.VMEM((NBUF, CH), jnp.int32),
            pltpu.VMEM((NBUF, CH, D), jnp.float32),
            pltpu.VMEM_SHARED((N, D), jnp.float32),
        ] + [pltpu.SemaphoreType.DMA] * (3 * NBUF),
    )
    def k(h_hbm, src_hbm, dst_hbm, z_hbm, out_hbm, sidx, didx, rows, acc,
          is0, is1, is2, is3,
          gs0, gs1, gs2, gs3,
          ss0, ss1, ss2, ss3):
        isems = (is0, is1, is2, is3)
        gsems = (gs0, gs1, gs2, gs3)
        ssems = (ss0, ss1, ss2, ss3)
        c = lax.axis_index("c")
        s = lax.axis_index("s")
        wid = c * NS + s
        base = wid * EPW

        def idx_start(i, b):
            off = pl.multiple_of(base + i * CH, 8)
            pltpu.async_copy(src_hbm.at[pl.ds(off, CH)], sidx.at[b], isems[b])
            pltpu.async_copy(dst_hbm.at[pl.ds(off, CH)], didx.at[b], isems[b])

        def idx_wait(b):
            pltpu.make_async_copy(src_hbm.at[pl.ds(0, CH)], sidx.at[b],
                                  isems[b]).wait()
            pltpu.make_async_copy(dst_hbm.at[pl.ds(0, CH)], didx.at[b],
                                  isems[b]).wait()

        pltpu.sync_copy(z_hbm.at[pl.ds(s * RPT, RPT)],
                        acc.at[pl.ds(s * RPT, RPT)])

        @pl.when(s == NS - 1)
        def _zero_tail():
            pltpu.sync_copy(z_hbm.at[pl.ds(NS * RPT, TAIL)],
                            acc.at[pl.ds(NS * RPT, TAIL)])

        plsc.subcore_barrier()

        idx_start(0, 0)
        idx_wait(0)
        pltpu.async_copy(h_hbm.at[sidx.at[0]], rows.at[0], gsems[0])
        pltpu.make_async_copy(h_hbm.at[sidx.at[0]], rows.at[0],
                              gsems[0]).wait()
        pltpu.async_copy(rows.at[0], acc.at[didx.at[0]], ssems[0], add=True)
        pltpu.make_async_copy(rows.at[0], acc.at[didx.at[0]],
                              ssems[0]).wait()

        for b in range(IA):
            idx_start(1 + b, b)
        for b in range(GA):
            idx_wait(b)
            pltpu.async_copy(h_hbm.at[sidx.at[b]], rows.at[b], gsems[b])

        @pl.loop(0, (NCHUNK - 1) // NBUF)
        def _grp(g):
            i0 = 1 + g * NBUF
            for b in range(NBUF):
                i = i0 + b
                pltpu.make_async_copy(h_hbm.at[sidx.at[b]], rows.at[b],
                                      gsems[b]).wait()
                pltpu.async_copy(rows.at[b], acc.at[didx.at[b]], ssems[b],
                                 add=True)
                bg = (b + GA) % NBUF

                @pl.when(i + GA < NCHUNK)
                def _issue_gather():
                    idx_wait(bg)

                    @pl.when(i >= NBUF - GA + 1)
                    def _wait_prev_scatter():
                        pltpu.make_async_copy(rows.at[bg], acc.at[didx.at[bg]],
                                              ssems[bg]).wait()

                    pltpu.async_copy(h_hbm.at[sidx.at[bg]], rows.at[bg],
                                     gsems[bg])

                bi = (b + IA) % NBUF

                @pl.when(i + IA < NCHUNK)
                def _issue_idx():
                    idx_start(i + IA, bi)

        for b in range(NBUF):
            pltpu.make_async_copy(rows.at[b], acc.at[didx.at[b]],
                                  ssems[b]).wait()

        plsc.subcore_barrier()
        pltpu.sync_copy(acc.at[pl.ds(s * RPT, RPT)],
                        out_hbm.at[pl.ds(c * N + s * RPT, RPT)])

        @pl.when(s == NS - 1)
        def _drain_tail():
            pltpu.sync_copy(acc.at[pl.ds(NS * RPT, TAIL)],
                            out_hbm.at[pl.ds(c * N + NS * RPT, TAIL)])

    return k(h, src, dst, zeros)


def _gin_layer_tc(h, msg2, eps, W1, b1, W2, b2, g, be):
    def body(eps_ref, h_ref, m_ref, w1_ref, b1_ref, w2_ref, b2_ref,
             g_ref, be_ref, o_ref):
        e = eps_ref[0]
        z = (1.0 + e) * h_ref[...] + m_ref[0] + m_ref[1]
        a = jnp.maximum(
            jnp.dot(z, w1_ref[...], preferred_element_type=jnp.float32)
            + b1_ref[...], 0.0)
        u = (jnp.dot(a, w2_ref[...], preferred_element_type=jnp.float32)
             + b2_ref[...])
        mu = jnp.mean(u, axis=0, keepdims=True)
        dvc = u - mu
        var = jnp.mean(dvc * dvc, axis=0, keepdims=True)
        o_ref[...] = jnp.maximum(
            dvc * lax.rsqrt(var + 1e-5) * g_ref[...] + be_ref[...], 0.0)

    return pl.pallas_call(
        body,
        out_shape=jax.ShapeDtypeStruct((N, HID), jnp.float32),
        in_specs=[pl.BlockSpec(memory_space=pltpu.SMEM)]
        + [pl.BlockSpec(memory_space=pltpu.VMEM)] * 8,
    )(eps.reshape(1), h, msg2, W1, b1.reshape(1, HID), W2,
      b2.reshape(1, HID), g.reshape(1, HID), be.reshape(1, HID))


def _gin_layer3_pool_tc(h, msg2, eps, W1, b1, W2, b2, g, be,
                        batchf, Wm, bm, Wv, bv):
    def body(eps_ref, h_ref, m_ref, w1_ref, b1_ref, w2_ref, b2_ref,
             g_ref, be_ref, b_ref, wm_ref, bm_ref, wv_ref, bv_ref,
             om_ref, ov_ref):
        e = eps_ref[0]
        z = (1.0 + e) * h_ref[...] + m_ref[0] + m_ref[1]
        a = jnp.maximum(
            jnp.dot(z, w1_ref[...], preferred_element_type=jnp.float32)
            + b1_ref[...], 0.0)
        u = (jnp.dot(a, w2_ref[...], preferred_element_type=jnp.float32)
             + b2_ref[...])
        mu = jnp.mean(u, axis=0, keepdims=True)
        dvc = u - mu
        var = jnp.mean(dvc * dvc, axis=0, keepdims=True)
        hf = jnp.maximum(
            dvc * lax.rsqrt(var + 1e-5) * g_ref[...] + be_ref[...], 0.0)
        iota_g = lax.broadcasted_iota(jnp.int32, (1, G), 1).astype(jnp.float32)
        oh = (b_ref[...] == iota_g).astype(jnp.float32)
        ones_col = jnp.ones((N, 1), dtype=jnp.float32)
        cnt = lax.dot_general(oh, ones_col, (((0,), (0,)), ((), ())),
                              preferred_element_type=jnp.float32)
        hgs = lax.dot_general(oh, hf, (((0,), (0,)), ((), ())),
                              preferred_element_type=jnp.float32)
        hg = hgs * (1.0 / jnp.clip(cnt, 1.0))
        om_ref[...] = (jnp.dot(hg, wm_ref[...],
                               preferred_element_type=jnp.float32)
                       + bm_ref[...])
        ov_ref[...] = (jnp.dot(hg, wv_ref[...],
                               preferred_element_type=jnp.float32)
                       + bv_ref[...])

    return pl.pallas_call(
        body,
        out_shape=(jax.ShapeDtypeStruct((G, L), jnp.float32),
                   jax.ShapeDtypeStruct((G, L), jnp.float32)),
        in_specs=[pl.BlockSpec(memory_space=pltpu.SMEM)]
        + [pl.BlockSpec(memory_space=pltpu.VMEM)] * 13,
    )(eps.reshape(1), h, msg2, W1, b1.reshape(1, HID), W2,
      b2.reshape(1, HID), g.reshape(1, HID), be.reshape(1, HID),
      batchf, Wm, bm.reshape(1, L), Wv, bv.reshape(1, L))


def kernel(x, edge_index, batch,
           eps0, W1_0, b1_0, W2_0, b2_0, g0, be0,
           eps1, W1_1, b1_1, W2_1, b2_1, g1, be1,
           eps2, W1_2, b1_2, W2_2, b2_2, g2, be2,
           Wm, bm, Wv, bv):
    src = edge_index[0]
    dst = edge_index[1]
    zeros = jnp.zeros((N, D), dtype=jnp.float32)
    batchf = batch.astype(jnp.float32).reshape(N, 1)

    h = x
    for eps, W1, b1, W2, b2, g, be in (
            (eps0, W1_0, b1_0, W2_0, b2_0, g0, be0),
            (eps1, W1_1, b1_1, W2_1, b2_1, g1, be1)):
        msg2 = _edge_messages(h, src, dst, zeros).reshape(NC, N, D)
        h = _gin_layer_tc(h, msg2, eps, W1, b1, W2, b2, g, be)

    msg2 = _edge_messages(h, src, dst, zeros).reshape(NC, N, D)
    return _gin_layer3_pool_tc(h, msg2, eps2, W1_2, b1_2, W2_2, b2_2, g2, be2,
                               batchf, Wm, bm, Wv, bv)
